# baseline (speedup 1.0000x reference)
import functools
import jax
import jax.numpy as jnp
from jax import lax
from jax.experimental import pallas as pl
from jax.experimental.pallas import tpu as pltpu

K = 32
TM = 64
TM2 = 128


def _enc_body(x_ref, we_ref, lat_ref):
    x = x_ref[...]
    we = we_ref[...]
    latent = lax.dot_general(
        x, we, (((1,), (1,)), ((), ())),
        preferred_element_type=jnp.float32,
    )
    latent = jnp.maximum(latent, 0.0)
    H = latent.shape[1]

    neg = jnp.float32(-jnp.inf)

    cmax = jnp.max(latent.reshape(TM, H // 128, 128), axis=2)

    def cstep(_, carry):
        work, _ = carry
        m = jnp.max(work, axis=1, keepdims=True)
        work = jnp.where(work >= m, neg, work)
        return work, m

    _, lower = lax.fori_loop(
        0, K, cstep, (cmax, jnp.zeros((TM, 1), jnp.float32))
    )
    rowmax = jnp.max(cmax, axis=1, keepdims=True)

    lo0 = jnp.maximum(lax.bitcast_convert_type(lower, jnp.int32), 0)
    hi0 = lax.bitcast_convert_type(rowmax, jnp.int32) + 1
    kf = jnp.float32(K)

    def cond(state):
        lo, hi, thr, done = state
        return jnp.sum(done) < TM

    def body(state):
        lo, hi, thr, done = state
        mid = lo + ((hi - lo) >> 1)
        midf = lax.bitcast_convert_type(mid, jnp.float32)
        cnt = jnp.sum(
            jnp.where(latent >= midf, 1.0, 0.0), axis=1, keepdims=True
        )
        p = cnt >= kf
        nlo = jnp.where(p, mid, lo)
        nhi = jnp.where(p, hi, mid)
        hit = cnt == kf
        live = done == 0
        thr = jnp.where(jnp.logical_and(live, hit), midf, thr)
        ndone = jnp.where(
            jnp.logical_or(hit, nhi - nlo <= 1), jnp.int32(1), done
        )
        lo = jnp.where(live, nlo, lo)
        hi = jnp.where(live, nhi, hi)
        return lo, hi, thr, ndone

    init = (
        lo0,
        hi0,
        jnp.full((TM, 1), neg, jnp.float32),
        jnp.where(hi0 - lo0 <= 1, jnp.int32(1), jnp.int32(0)),
    )
    lo_f, _, thr_f, done_f = lax.while_loop(cond, body, init)
    thresh = jnp.where(
        thr_f > neg, thr_f, lax.bitcast_convert_type(lo_f, jnp.float32)
    )

    lat_ref[...] = jnp.where(latent >= thresh, latent, 0.0)


def _dec_body(lat_ref, wd_ref, rec_ref):
    wd = wd_ref[...]
    rec_ref[...] = lax.dot_general(
        lat_ref[...], wd, (((1,), (1,)), ((), ())),
        preferred_element_type=jnp.float32,
    )


def kernel(x, W_enc, W_dec):
    N, D = x.shape
    H = W_enc.shape[0]

    latent_sparse = pl.pallas_call(
        _enc_body,
        grid=(N // TM,),
        in_specs=[
            pl.BlockSpec((TM, D), lambda i: (i, 0)),
            pl.BlockSpec((H, D), lambda i: (0, 0)),
        ],
        out_specs=pl.BlockSpec((TM, H), lambda i: (i, 0)),
        out_shape=jax.ShapeDtypeStruct((N, H), jnp.float32),
        compiler_params=pltpu.CompilerParams(
            dimension_semantics=("arbitrary",),
        ),
    )(x, W_enc)

    recon = pl.pallas_call(
        _dec_body,
        grid=(N // TM2,),
        in_specs=[
            pl.BlockSpec((TM2, H), lambda i: (i, 0)),
            pl.BlockSpec((D, H), lambda i: (0, 0)),
        ],
        out_specs=pl.BlockSpec((TM2, D), lambda i: (i, 0)),
        out_shape=jax.ShapeDtypeStruct((N, D), jnp.float32),
        compiler_params=pltpu.CompilerParams(
            dimension_semantics=("arbitrary",),
        ),
    )(latent_sparse, W_dec)

    return (latent_sparse, recon)
